# baseline (speedup 1.0000x reference)
import jax
import jax.numpy as jnp
from jax import lax
from jax.experimental import pallas as pl
from jax.experimental.pallas import tpu as pltpu
from jax.experimental.pallas import tpu_sc as plsc

L = 8192
H = 16
C = 128
NB = 512
NC = 2
NS = 16

HALF = L // NC
CHUNK = 128
NCHUNK = HALF // CHUNK
GRP = 16
NGRP = CHUNK // GRP


def _grpe_body(x_hbm, rp_hbm, tab_hbm, out_hbm, tab_v, idx_v, xv0, xv1,
               out_v, sem0, sem1):
    cid = lax.axis_index("c")
    sid = lax.axis_index("s")
    h = sid
    base = cid * HALF

    pltpu.sync_copy(tab_hbm.at[:, h, :], tab_v)
    pltpu.sync_copy(rp_hbm.at[pl.ds(base, HALF)], idx_v)

    def chunk_copy(k, b):
        sem = sem0 if b == 0 else sem1
        buf = xv0 if b == 0 else xv1
        return pltpu.make_async_copy(
            x_hbm.at[pl.ds(base + k * CHUNK, CHUNK), h, :], buf, sem)

    chunk_copy(0, 0).start()
    chunk_copy(1, 1).start()

    lane = jnp.arange(GRP, dtype=jnp.int32)
    zeros = jnp.zeros((GRP,), jnp.float32)

    def compute_chunk(k, b):
        buf = xv0 if b == 0 else xv1
        def grp_body(g, carry):
            t0 = k * CHUNK + g * GRP
            bv = idx_v[pl.ds(t0, GRP)]
            rowv = lane + g * GRP
            def cblk(cb, accs, wrap):
                cbase = cb * GRP
                for j in range(GRP):
                    col = lane + (j + cbase)
                    if wrap:
                        col = col & (C - 1)
                    gx = plsc.load_gather(buf, [rowv, col])
                    gw = plsc.load_gather(tab_v, [bv, col])
                    accs = tuple(
                        accs[i] + gx * gw if i == j % 4 else accs[i]
                        for i in range(4))
                return accs
            accs = lax.fori_loop(0, C // GRP - 1,
                                 lambda cb, a: cblk(cb, a, False),
                                 (zeros, zeros, zeros, zeros))
            accs = cblk(C // GRP - 1, accs, True)
            out_v[pl.ds(t0, GRP)] = (accs[0] + accs[1]) + (accs[2] + accs[3])
            return carry
        lax.fori_loop(0, NGRP, grp_body, 0)

    def outer(i, carry):
        for b in range(2):
            k = i * 2 + b
            chunk_copy(k, b).wait()
            compute_chunk(k, b)
            @pl.when(k + 2 < NCHUNK)
            def _():
                chunk_copy(k + 2, b).start()
        return carry

    lax.fori_loop(0, NCHUNK // 2, outer, 0)

    pltpu.sync_copy(out_v, out_hbm.at[h, pl.ds(base, HALF)])


_grpe_sc = pl.kernel(
    _grpe_body,
    out_type=jax.ShapeDtypeStruct((H, L), jnp.float32),
    mesh=plsc.VectorSubcoreMesh(
        core_axis_name="c", subcore_axis_name="s",
        num_cores=NC, num_subcores=NS),
    compiler_params=pltpu.CompilerParams(needs_layout_passes=False),
    scratch_types=[
        pltpu.VMEM((NB, C), jnp.float32),
        pltpu.VMEM((HALF,), jnp.int32),
        pltpu.VMEM((CHUNK, C), jnp.float32),
        pltpu.VMEM((CHUNK, C), jnp.float32),
        pltpu.VMEM((HALF,), jnp.float32),
        pltpu.SemaphoreType.DMA,
        pltpu.SemaphoreType.DMA,
    ],
)


def kernel(x, rp_bucket, lookup_table_weight):
    out_t = _grpe_sc(x, rp_bucket, lookup_table_weight)
    return out_t.T


# speedup vs baseline: 1.2275x; 1.2275x over previous
import jax
import jax.numpy as jnp
from jax import lax
from jax.experimental import pallas as pl
from jax.experimental.pallas import tpu as pltpu
from jax.experimental.pallas import tpu_sc as plsc

L = 8192
H = 16
C = 128
NB = 512
NC = 2
NS = 16

HALF = L // NC
CHUNK = 128
NCHUNK = HALF // CHUNK
GRP = 16
NGRP = CHUNK // GRP


def _grpe_body(x_hbm, rp_hbm, tab_hbm, out_hbm, tab_v, idx_v, xv0, xv1,
               out_v, sem0, sem1):
    cid = lax.axis_index("c")
    sid = lax.axis_index("s")
    h = sid
    base = cid * HALF

    pltpu.sync_copy(tab_hbm.at[:, h, :], tab_v)
    pltpu.sync_copy(rp_hbm.at[pl.ds(base, HALF)], idx_v)

    def chunk_copy(k, b):
        sem = sem0 if b == 0 else sem1
        buf = xv0 if b == 0 else xv1
        return pltpu.make_async_copy(
            x_hbm.at[pl.ds(base + k * CHUNK, CHUNK), h, :], buf, sem)

    chunk_copy(0, 0).start()
    chunk_copy(1, 1).start()

    lane = jnp.arange(GRP, dtype=jnp.int32)
    zeros = jnp.zeros((GRP,), jnp.float32)

    def compute_chunk(k, b):
        buf = xv0 if b == 0 else xv1
        def grp_body(g, carry):
            t0 = k * CHUNK + g * GRP
            bv = idx_v[pl.ds(t0, GRP)]
            rowv = lane + g * GRP
            def cblk_body(cb, accs):
                cbase = cb * GRP
                for j in range(GRP):
                    col = (lane + (j + cbase)) & (C - 1)
                    gx = plsc.load_gather(buf, [rowv, col])
                    gw = plsc.load_gather(tab_v, [bv, col])
                    accs = tuple(
                        accs[i] + gx * gw if i == j % 2 else accs[i]
                        for i in range(2))
                return accs
            accs = lax.fori_loop(0, C // GRP, cblk_body, (zeros, zeros))
            out_v[pl.ds(t0, GRP)] = accs[0] + accs[1]
            return carry
        lax.fori_loop(0, NGRP, grp_body, 0)

    def outer(i, carry):
        for b in range(2):
            k = i * 2 + b
            chunk_copy(k, b).wait()
            compute_chunk(k, b)
            @pl.when(k + 2 < NCHUNK)
            def _():
                chunk_copy(k + 2, b).start()
        return carry

    lax.fori_loop(0, NCHUNK // 2, outer, 0)

    pltpu.sync_copy(out_v, out_hbm.at[h, pl.ds(base, HALF)])


_grpe_sc = pl.kernel(
    _grpe_body,
    out_type=jax.ShapeDtypeStruct((H, L), jnp.float32),
    mesh=plsc.VectorSubcoreMesh(
        core_axis_name="c", subcore_axis_name="s",
        num_cores=NC, num_subcores=NS),
    compiler_params=pltpu.CompilerParams(needs_layout_passes=False),
    scratch_types=[
        pltpu.VMEM((NB, C), jnp.float32),
        pltpu.VMEM((HALF,), jnp.int32),
        pltpu.VMEM((CHUNK, C), jnp.float32),
        pltpu.VMEM((CHUNK, C), jnp.float32),
        pltpu.VMEM((HALF,), jnp.float32),
        pltpu.SemaphoreType.DMA,
        pltpu.SemaphoreType.DMA,
    ],
)


def kernel(x, rp_bucket, lookup_table_weight):
    out_t = _grpe_sc(x, rp_bucket, lookup_table_weight)
    return out_t.T
